# baseline (speedup 1.0000x reference)
import jax
import jax.numpy as jnp
from jax.experimental import pallas as pl
from jax.experimental.pallas import tpu as pltpu

B, F_IN, F_OUT = 64, 4096, 4096
OB = 512


def _mm_body(x_ref, w_ref, m_ref, o_ref):
    wm = w_ref[...] * m_ref[...].astype(jnp.float32)
    o_ref[...] = jax.lax.dot_general(
        x_ref[...], wm, (((1,), (1,)), ((), ())),
        preferred_element_type=jnp.float32)


def kernel(x, weight, mask):
    grid = (F_OUT // OB,)
    return pl.pallas_call(
        _mm_body,
        grid=grid,
        in_specs=[
            pl.BlockSpec((B, F_IN), lambda o: (0, 0)),
            pl.BlockSpec((OB, F_IN), lambda o: (o, 0)),
            pl.BlockSpec((OB, F_IN), lambda o: (o, 0)),
        ],
        out_specs=pl.BlockSpec((B, OB), lambda o: (0, o)),
        out_shape=jax.ShapeDtypeStruct((B, F_OUT), jnp.float32),
        compiler_params=pltpu.CompilerParams(
            dimension_semantics=("arbitrary",)),
    )(x, weight, mask)


# speedup vs baseline: 1.0427x; 1.0427x over previous
import jax
import jax.numpy as jnp
from jax.experimental import pallas as pl
from jax.experimental.pallas import tpu as pltpu

B, F_IN, F_OUT = 64, 4096, 4096
OB = 512
NS = 4
OBS = OB // NS


def _mm_body(x_ref, *refs):
    w_refs = refs[:NS]
    m_refs = refs[NS:2 * NS]
    o_ref = refs[2 * NS]
    xv = x_ref[...]
    for r in range(NS):
        wm = w_refs[r][...] * m_refs[r][...].astype(jnp.float32)
        o_ref[:, r * OBS:(r + 1) * OBS] = jax.lax.dot_general(
            xv, wm, (((1,), (1,)), ((), ())),
            preferred_element_type=jnp.float32)


def kernel(x, weight, mask):
    grid = (F_OUT // OB,)
    w_specs = [
        pl.BlockSpec((OBS, F_IN), lambda o, r=r: (NS * o + r, 0))
        for r in range(NS)
    ]
    m_specs = [
        pl.BlockSpec((OBS, F_IN), lambda o, r=r: (NS * o + r, 0))
        for r in range(NS)
    ]
    return pl.pallas_call(
        _mm_body,
        grid=grid,
        in_specs=[pl.BlockSpec((B, F_IN), lambda o: (0, 0))]
        + w_specs + m_specs,
        out_specs=pl.BlockSpec((B, OB), lambda o: (0, o)),
        out_shape=jax.ShapeDtypeStruct((B, F_OUT), jnp.float32),
        compiler_params=pltpu.CompilerParams(
            dimension_semantics=("arbitrary",)),
    )(x, *([weight] * NS), *([mask] * NS))
